# baseline (speedup 1.0000x reference)
import jax
import jax.numpy as jnp
from jax import lax
from jax.experimental import pallas as pl
from jax.experimental.pallas import tpu as pltpu
from jax.experimental.pallas import tpu_sc as plsc

_NC = 2
_NS = 16
_NW = _NC * _NS
_L = 16

_B = 16384
_D = 64
_XW = 16
_BPW = _B // _NW
_CH = 128
_NCH = _BPW // _CH
_FROWS = 112
_ZROW = 104


def _fused_body(wl_ref, wc_ref, n0_ref, o_ref):
    k4 = lax.broadcasted_iota(jnp.int32, (_FROWS, 4), 0)
    j4 = lax.broadcasted_iota(jnp.int32, (_FROWS, 4), 1)
    a = ((k4 // 26) == j4).astype(jnp.float32)
    k26 = lax.broadcasted_iota(jnp.int32, (_FROWS, 26), 0)
    j26 = lax.broadcasted_iota(jnp.int32, (_FROWS, 26), 1)
    b = (((k26 % 26) == j26) & (k26 < 104)).astype(jnp.float32)
    live = (lax.broadcasted_iota(jnp.int32, (_FROWS, 1), 0) < 104)
    o_ref[...] = (
        jnp.dot(a, wl_ref[...], preferred_element_type=jnp.float32,
                precision=lax.Precision.HIGHEST)
        + jnp.dot(b, wc_ref[...], preferred_element_type=jnp.float32,
                  precision=lax.Precision.HIGHEST)
        + jnp.where(live, n0_ref[...], 0.0)
    )


def _build_fused(wl, wc, n0):
    return pl.pallas_call(
        _fused_body,
        out_shape=jax.ShapeDtypeStruct((_FROWS, _D), jnp.float32),
    )(wl, wc, n0)


def _sc_body(w_hbm, nm_hbm, xs_hbm, idx_hbm, nu_hbm, fu_hbm, out_hbm,
             idx_v, nu_v, g1_v, g2_v, nm_v, w_v, n_v, xs_v, fu_v, out_v, sem):
    wid = lax.axis_index("s") * _NC + lax.axis_index("c")
    base = wid * _BPW

    with jax.named_scope("stage"):
        pltpu.sync_copy(idx_hbm.at[pl.ds(base, _BPW)], idx_v)
        pltpu.sync_copy(nu_hbm, nu_v)
        fdesc = pltpu.async_copy(fu_hbm, fu_v, sem)

    nu = nu_v[...]
    with jax.named_scope("pass1"):
        for i in range(_BPW // _L):
            sl = pl.ds(i * _L, _L)
            idxc = idx_v[sl]
            item = idxc >= nu
            g1_v[sl] = jnp.where(item, idxc + 30, idxc)
            g2_v[sl] = jnp.where(item, idxc + 30, 0)
            nm_v[sl] = jnp.where(item, 1.0, 0.0)

    with jax.named_scope("gather"):
        descs = []
        for j in range(_NCH):
            sl = pl.ds(j * _CH, _CH)
            descs.append(pltpu.async_copy(w_hbm.at[g1_v.at[sl]], w_v.at[sl], sem))
            descs.append(pltpu.async_copy(nm_hbm.at[g2_v.at[sl]], n_v.at[sl], sem))
            descs.append(pltpu.async_copy(xs_hbm.at[g1_v.at[sl]], xs_v.at[sl], sem))
        fdesc.wait()
        for d in descs:
            d.wait()

    with jax.named_scope("combine"):
        @plsc.parallel_loop(0, _BPW // _L)
        def _combine(ci):
            nm16 = nm_v[pl.ds(ci * _L, _L)]
            for k in range(_L):
                bb = ci * _L + k
                tv = xs_v[bb, pl.ds(0, _L)]
                nm_s = nm16[k]
                lvl = tv[0].astype(jnp.int32)
                ins = tv[1].astype(jnp.int32)
                frow = jnp.where(nm_s > 0.5, _ZROW, lvl * 26 + ins)
                for j in range(_D // _L):
                    sl = pl.ds(j * _L, _L)
                    out_v[bb, sl] = (
                        w_v[bb, sl]
                        + nm_s * n_v[bb, sl]
                        + fu_v[frow, sl]
                    )

    with jax.named_scope("writeback"):
        pltpu.sync_copy(out_v, out_hbm.at[pl.ds(base, _BPW)])


@jax.jit
def _sc_lookup(weight, name_emb, xside, idx, nu_vec, fused):
    mesh = plsc.VectorSubcoreMesh(core_axis_name="c", subcore_axis_name="s")
    return pl.kernel(
        _sc_body,
        out_type=jax.ShapeDtypeStruct((_B, _D), jnp.float32),
        mesh=mesh,
        compiler_params=pltpu.CompilerParams(use_tc_tiling_on_sc=False),
        scratch_types=[
            pltpu.VMEM((_BPW,), jnp.int32),
            pltpu.VMEM((_L,), jnp.int32),
            pltpu.VMEM((_BPW,), jnp.int32),
            pltpu.VMEM((_BPW,), jnp.int32),
            pltpu.VMEM((_BPW,), jnp.float32),
            pltpu.VMEM((_BPW, _D), jnp.float32),
            pltpu.VMEM((_BPW, _D), jnp.float32),
            pltpu.VMEM((_BPW, _XW), jnp.float32),
            pltpu.VMEM((_FROWS, _D), jnp.float32),
            pltpu.VMEM((_BPW, _D), jnp.float32),
            pltpu.SemaphoreType.DMA,
        ],
    )(weight, name_emb, xside, idx, nu_vec, fused)


def kernel(x, idx, num_users, weight, name_emb):
    x = x.astype(jnp.int32)
    idx = idx.astype(jnp.int32)
    nu = jnp.asarray(num_users, jnp.int32)
    wl = lax.dynamic_slice_in_dim(weight, nu, 4, axis=0)
    wc = lax.dynamic_slice_in_dim(weight, nu + 4, 26, axis=0)
    fused = _build_fused(wl, wc, name_emb[0:1])
    nrows = weight.shape[0]
    xside = jnp.zeros((nrows, _XW), jnp.float32)
    xside = lax.dynamic_update_slice(
        xside, x[:, 1:3].astype(jnp.float32), (0, 0))
    nu_vec = jnp.full((_L,), nu, jnp.int32)
    return _sc_lookup(weight, name_emb, xside, idx, nu_vec, fused)


# speedup vs baseline: 3.0733x; 3.0733x over previous
import jax
import jax.numpy as jnp
from jax import lax
from jax.experimental import pallas as pl
from jax.experimental.pallas import tpu as pltpu
from jax.experimental.pallas import tpu_sc as plsc

_NC = 2
_NS = 16
_NW = _NC * _NS
_L = 16

_B = 16384
_D = 64
_CW = 2 * _D
_BPW = _B // _NW
_CH = 128
_NCH = _BPW // _CH
_FROWS = 112
_ZROW = 104


def _fused_body(wl_ref, wc_ref, n0_ref, o_ref):
    k4 = lax.broadcasted_iota(jnp.int32, (_FROWS, 4), 0)
    j4 = lax.broadcasted_iota(jnp.int32, (_FROWS, 4), 1)
    a = ((k4 // 26) == j4).astype(jnp.float32)
    k26 = lax.broadcasted_iota(jnp.int32, (_FROWS, 26), 0)
    j26 = lax.broadcasted_iota(jnp.int32, (_FROWS, 26), 1)
    b = (((k26 % 26) == j26) & (k26 < 104)).astype(jnp.float32)
    live = (lax.broadcasted_iota(jnp.int32, (_FROWS, 1), 0) < 104)
    left = (
        jnp.dot(a, wl_ref[...], preferred_element_type=jnp.float32,
                precision=lax.Precision.HIGHEST)
        + jnp.dot(b, wc_ref[...], preferred_element_type=jnp.float32,
                  precision=lax.Precision.HIGHEST)
        + jnp.where(live, n0_ref[...], 0.0)
    )
    o_ref[...] = jnp.concatenate(
        [left, jnp.zeros((_FROWS, _D), jnp.float32)], axis=1)


def _build_fused(wl, wc, n0):
    return pl.pallas_call(
        _fused_body,
        out_shape=jax.ShapeDtypeStruct((_FROWS, _CW), jnp.float32),
    )(wl, wc, n0)


def _idx_body(x1_hbm, x2_hbm, idx_hbm, nu_hbm, g1_hbm, fr_hbm,
              idx_v, nu_v, lv_v, in_v, g1_v, fr_v, sem):
    wid = lax.axis_index("s") * _NC + lax.axis_index("c")
    base = wid * _BPW

    with jax.named_scope("stage_a"):
        pltpu.sync_copy(idx_hbm.at[pl.ds(base, _BPW)], idx_v)
        pltpu.sync_copy(nu_hbm, nu_v)

    with jax.named_scope("gather_x"):
        descs = []
        for j in range(_NCH):
            sl = pl.ds(j * _CH, _CH)
            descs.append(
                pltpu.async_copy(x1_hbm.at[idx_v.at[sl]], lv_v.at[sl], sem))
            descs.append(
                pltpu.async_copy(x2_hbm.at[idx_v.at[sl]], in_v.at[sl], sem))
        for d in descs:
            d.wait()

    nu = nu_v[...]
    with jax.named_scope("index_math"):
        for i in range(_BPW // _L):
            sl = pl.ds(i * _L, _L)
            idxc = idx_v[sl]
            item = idxc >= nu
            g1_v[sl] = jnp.where(item, idxc + 30, idxc)
            fr_v[sl] = jnp.where(item, _ZROW, lv_v[sl] * 26 + in_v[sl])

    with jax.named_scope("writeback_a"):
        pltpu.sync_copy(g1_v, g1_hbm.at[pl.ds(base, _BPW)])
        pltpu.sync_copy(fr_v, fr_hbm.at[pl.ds(base, _BPW)])


def _sc_index(x1, x2, idx, nu_vec):
    mesh = plsc.VectorSubcoreMesh(core_axis_name="c", subcore_axis_name="s")
    return pl.kernel(
        _idx_body,
        out_type=(jax.ShapeDtypeStruct((_B,), jnp.int32),
                  jax.ShapeDtypeStruct((_B,), jnp.int32)),
        mesh=mesh,
        compiler_params=pltpu.CompilerParams(use_tc_tiling_on_sc=False),
        scratch_types=[
            pltpu.VMEM((_BPW,), jnp.int32),
            pltpu.VMEM((_L,), jnp.int32),
            pltpu.VMEM((_BPW,), jnp.int32),
            pltpu.VMEM((_BPW,), jnp.int32),
            pltpu.VMEM((_BPW,), jnp.int32),
            pltpu.VMEM((_BPW,), jnp.int32),
            pltpu.SemaphoreType.DMA,
        ],
    )(x1, x2, idx, nu_vec)


def _main_body(c_hbm, g1_hbm, fr_hbm, fu_hbm, out_hbm,
               g1_v, fr_v, c_v, fu_v, out_v, sem):
    wid = lax.axis_index("s") * _NC + lax.axis_index("c")
    base = wid * _BPW

    with jax.named_scope("stage_b"):
        pltpu.sync_copy(g1_hbm.at[pl.ds(base, _BPW)], g1_v)
        pltpu.sync_copy(fr_hbm.at[pl.ds(base, _BPW)], fr_v)
        fdesc = pltpu.async_copy(fu_hbm, fu_v, sem)
        fdesc.wait()

    for h in range(2):
        hof = h * (_BPW // 2)
        with jax.named_scope("gather"):
            descs = []
            for j in range(_NCH // 2):
                sl = pl.ds(hof + j * _CH, _CH)
                dl = pl.ds(j * _CH, _CH)
                descs.append(
                    pltpu.async_copy(c_hbm.at[g1_v.at[sl]], c_v.at[dl], sem))
            for d in descs:
                d.wait()

        with jax.named_scope("combine"):
            @plsc.parallel_loop(0, _BPW // (2 * _L))
            def _combine(ci):
                fr16 = fr_v[pl.ds(hof + ci * _L, _L)]
                for k in range(_L):
                    bb = ci * _L + k
                    fr_s = fr16[k]
                    nm_s = jnp.where(fr_s == _ZROW, 1.0, 0.0)
                    for j in range(_D // _L):
                        sl = pl.ds(j * _L, _L)
                        out_v[bb, sl] = (
                            c_v[bb, sl]
                            + nm_s * c_v[bb, pl.ds(_D + j * _L, _L)]
                            + fu_v[fr_s, sl]
                        )

        with jax.named_scope("writeback"):
            pltpu.sync_copy(out_v, out_hbm.at[pl.ds(base + hof, _BPW // 2)])


def _sc_main(ctab, g1, fr, fused):
    mesh = plsc.VectorSubcoreMesh(core_axis_name="c", subcore_axis_name="s")
    return pl.kernel(
        _main_body,
        out_type=jax.ShapeDtypeStruct((_B, _D), jnp.float32),
        mesh=mesh,
        scratch_types=[
            pltpu.VMEM((_BPW,), jnp.int32),
            pltpu.VMEM((_BPW,), jnp.int32),
            pltpu.VMEM((_BPW // 2, _CW), jnp.float32),
            pltpu.VMEM((_FROWS, _CW), jnp.float32),
            pltpu.VMEM((_BPW // 2, _D), jnp.float32),
            pltpu.SemaphoreType.DMA,
        ],
    )(ctab, g1, fr, fused)


@jax.jit
def _run(x1, x2, idx, nu_vec, ctab, fused):
    g1, fr = _sc_index(x1, x2, idx, nu_vec)
    return _sc_main(ctab, g1, fr, fused)


def kernel(x, idx, num_users, weight, name_emb):
    x = x.astype(jnp.int32)
    idx = idx.astype(jnp.int32)
    nu = jnp.asarray(num_users, jnp.int32)
    wl = lax.dynamic_slice_in_dim(weight, nu, 4, axis=0)
    wc = lax.dynamic_slice_in_dim(weight, nu + 4, 26, axis=0)
    fused = _build_fused(wl, wc, name_emb[0:1])
    ctab = jnp.concatenate([weight, name_emb], axis=1)
    x1 = jnp.asarray(x[:, 1])
    x2 = jnp.asarray(x[:, 2])
    nu_vec = jnp.full((_L,), nu, jnp.int32)
    return _run(x1, x2, idx, nu_vec, ctab, fused)
